# baseline (speedup 1.0000x reference)
import functools

import numpy as np
import jax
import jax.numpy as jnp
from jax import lax
from jax.experimental import pallas as pl
from jax.experimental.pallas import tpu as pltpu
from jax.experimental.pallas import tpu_sc as plsc

_VOCAB = 100000
_D = 128
_SEQ = 200
_BATCH = 1024
_NW = 32
_Q = 4
_P = 25
_CHUNK = _Q * _P
_NPB = _SEQ // _P
_SPW = _BATCH // _NW
_NQG = _SPW // _Q
_NCH = _NQG * _NPB
_NBUF = 4
_SCALE = float(np.sqrt(float(_D)))


def _pos_table() -> np.ndarray:
    d = np.arange(_D)
    even = (d % 2 == 0).astype(np.float64)
    odd = (d % 2 == 1).astype(np.float64)
    rate = 1.0 / (10000.0 ** (d[np.newaxis, :] / _D))
    rads = np.arange(_SEQ)[:, np.newaxis] * rate
    return (np.sin(rads) * even + np.cos(rads) * odd).astype(np.float32)


_POS = _pos_table()

_mesh = plsc.VectorSubcoreMesh(core_axis_name="c", subcore_axis_name="s")


@functools.partial(
    pl.kernel,
    mesh=_mesh,
    out_type=jax.ShapeDtypeStruct((_BATCH, _NPB, _P, _D), jnp.float32),
    scratch_types=[
        pltpu.VMEM((_NCH, _CHUNK), jnp.int32),
        pltpu.VMEM((_SEQ, _D), jnp.float32),
        pltpu.VMEM((_NBUF, _CHUNK, _D), jnp.float32),
        pltpu.SemaphoreType.DMA,
        pltpu.SemaphoreType.DMA,
        pltpu.SemaphoreType.DMA,
        pltpu.SemaphoreType.DMA,
        pltpu.SemaphoreType.DMA,
        pltpu.SemaphoreType.DMA,
        pltpu.SemaphoreType.DMA,
        pltpu.SemaphoreType.DMA,
    ],
)
def _emb_lookup(idx_hbm, tab_hbm, pos_hbm, out_hbm, idx_v, pos_v, buf,
                gs0, gs1, gs2, gs3, ss0, ss1, ss2, ss3):
    gsems = (gs0, gs1, gs2, gs3)
    ssems = (ss0, ss1, ss2, ss3)
    wid = lax.axis_index("s") * 2 + lax.axis_index("c")
    b_base = wid * _SPW
    pltpu.sync_copy(pos_hbm, pos_v)
    pltpu.sync_copy(idx_hbm.at[wid], idx_v)

    def issue_gather(g, s):
        pltpu.async_copy(tab_hbm.at[idx_v.at[g]], buf.at[s], gsems[s])

    def wait_gather(s):
        for q in range(_Q):
            pltpu.make_async_copy(out_hbm.at[0, 0], buf.at[s, pl.ds(0, _P)],
                                  gsems[s]).wait()

    def issue_store(g, s):
        qg = g // _NPB
        pblk = g % _NPB
        b0 = b_base + qg * _Q
        for q in range(_Q):
            pltpu.async_copy(buf.at[s, pl.ds(q * _P, _P)],
                             out_hbm.at[b0 + q, pblk], ssems[s])

    def wait_store(s):
        for _ in range(_Q):
            pltpu.make_async_copy(buf.at[s, pl.ds(0, _P)], out_hbm.at[0, 0],
                                  ssems[s]).wait()

    def compute(s, g):
        po = (g % _NPB) * _P

        def p_body(p, c):
            pr = po + p
            for j in range(_D // 16):
                sl = pl.ds(j * 16, 16)
                pv = pos_v[pr, sl]
                for q in range(_Q):
                    r = q * _P + p
                    buf[s, r, sl] = buf[s, r, sl] * _SCALE + pv
            return c

        lax.fori_loop(0, _P, p_body, 0)

    for b in range(_NBUF - 1):
        issue_gather(b, b)

    wait_gather(0)
    compute(0, 0)
    issue_gather(_NBUF - 1, _NBUF - 1)
    issue_store(0, 0)
    for b in range(1, _NBUF):
        wait_gather(b)
        compute(b, b)
        wait_store(b - 1)
        issue_gather(b + _NBUF - 1, b - 1)
        issue_store(b, b)

    def block_body(it, carry):
        g0 = it * _NBUF
        for b in range(_NBUF):
            g = g0 + b
            wait_gather(b)
            compute(b, g)
            wait_store((b - 1) % _NBUF)
            issue_gather(g + _NBUF - 1, (b - 1) % _NBUF)
            issue_store(g, b)
        return carry

    lax.fori_loop(1, _NCH // _NBUF - 1, block_body, 0)

    g0 = _NCH - _NBUF
    wait_gather(0)
    compute(0, g0)
    wait_store(_NBUF - 1)
    issue_gather(g0 + _NBUF - 1, _NBUF - 1)
    issue_store(g0, 0)
    for b in range(1, _NBUF):
        wait_gather(b)
        compute(b, g0 + b)
        wait_store(b - 1)
        issue_store(g0 + b, b)
    wait_store(_NBUF - 1)


def kernel(x, embedding_table):
    idx = (x.reshape(_NW, _NQG, _Q, _NPB, _P)
           .transpose(0, 1, 3, 2, 4)
           .reshape(_NW, _NCH, _CHUNK)
           .astype(jnp.int32))
    pos = jnp.asarray(_POS)
    out = _emb_lookup(idx, embedding_table, pos)
    return out.reshape(_BATCH, _SEQ, _D)


# speedup vs baseline: 1.4890x; 1.4890x over previous
import functools

import numpy as np
import jax
import jax.numpy as jnp
from jax import lax
from jax.experimental import pallas as pl
from jax.experimental.pallas import tpu as pltpu
from jax.experimental.pallas import tpu_sc as plsc

_VOCAB = 100000
_D = 128
_SEQ = 200
_BATCH = 1024
_NW = 32
_CHUNK = 100
_TOK = _BATCH * _SEQ
_TPW = _TOK // _NW
_NCH = _TPW // _CHUNK
_NBUF = 4
_SCALE = float(np.sqrt(float(_D)))


def _pos_table() -> np.ndarray:
    d = np.arange(_D)
    even = (d % 2 == 0).astype(np.float64)
    odd = (d % 2 == 1).astype(np.float64)
    rate = 1.0 / (10000.0 ** (d[np.newaxis, :] / _D))
    rads = np.arange(_SEQ)[:, np.newaxis] * rate
    return (np.sin(rads) * even + np.cos(rads) * odd).astype(np.float32)


_POS = _pos_table()

_mesh = plsc.VectorSubcoreMesh(core_axis_name="c", subcore_axis_name="s")


@functools.partial(
    pl.kernel,
    mesh=_mesh,
    out_type=jax.ShapeDtypeStruct((_NW, _NCH, _CHUNK, _D), jnp.float32),
    scratch_types=[
        pltpu.VMEM((_NCH, _CHUNK), jnp.int32),
        pltpu.VMEM((_SEQ, _D), jnp.float32),
        pltpu.VMEM((_NBUF, _CHUNK, _D), jnp.float32),
        pltpu.SemaphoreType.DMA,
        pltpu.SemaphoreType.DMA,
        pltpu.SemaphoreType.DMA,
        pltpu.SemaphoreType.DMA,
        pltpu.SemaphoreType.DMA,
        pltpu.SemaphoreType.DMA,
        pltpu.SemaphoreType.DMA,
        pltpu.SemaphoreType.DMA,
    ],
)
def _emb_lookup(idx_hbm, tab_hbm, pos_hbm, out_hbm, idx_v, pos_v, buf,
                gs0, gs1, gs2, gs3, ss0, ss1, ss2, ss3):
    gsems = (gs0, gs1, gs2, gs3)
    ssems = (ss0, ss1, ss2, ss3)
    wid = lax.axis_index("s") * 2 + lax.axis_index("c")
    pltpu.sync_copy(pos_hbm, pos_v)
    pltpu.sync_copy(idx_hbm.at[wid], idx_v)

    def issue_gather(g, s):
        pltpu.async_copy(tab_hbm.at[idx_v.at[g]], buf.at[s], gsems[s])

    def wait_gather(s):
        pltpu.make_async_copy(out_hbm.at[0, 0], buf.at[s], gsems[s]).wait()

    def issue_store(g, s):
        pltpu.async_copy(buf.at[s], out_hbm.at[wid, g], ssems[s])

    def wait_store(s):
        pltpu.make_async_copy(buf.at[s], out_hbm.at[wid, 0], ssems[s]).wait()

    def compute(s, po):
        def row_body(r, c):
            for j in range(_D // 16):
                sl = pl.ds(j * 16, 16)
                buf[s, r, sl] = buf[s, r, sl] * _SCALE + pos_v[po + r, sl]
            return c

        lax.fori_loop(0, _CHUNK, row_body, 0)

    for b in range(_NBUF - 1):
        issue_gather(b, b)

    wait_gather(0)
    compute(0, 0)
    issue_gather(_NBUF - 1, _NBUF - 1)
    issue_store(0, 0)
    for b in range(1, _NBUF):
        wait_gather(b)
        compute(b, (b % 2) * _CHUNK)
        wait_store(b - 1)
        issue_gather(b + _NBUF - 1, b - 1)
        issue_store(b, b)

    def block_body(it, carry):
        g0 = it * _NBUF
        for b in range(_NBUF):
            g = g0 + b
            wait_gather(b)
            compute(b, (b % 2) * _CHUNK)
            wait_store((b - 1) % _NBUF)
            issue_gather(g + _NBUF - 1, (b - 1) % _NBUF)
            issue_store(g, b)
        return carry

    lax.fori_loop(1, _NCH // _NBUF - 1, block_body, 0)

    g0 = _NCH - _NBUF
    wait_gather(0)
    compute(0, 0)
    wait_store(_NBUF - 1)
    issue_gather(g0 + _NBUF - 1, _NBUF - 1)
    issue_store(g0, 0)
    for b in range(1, _NBUF):
        wait_gather(b)
        compute(b, (b % 2) * _CHUNK)
        wait_store(b - 1)
        issue_store(g0 + b, b)
    wait_store(_NBUF - 1)


def kernel(x, embedding_table):
    idx = x.reshape(_NW, _NCH, _CHUNK).astype(jnp.int32)
    pos = jnp.asarray(_POS)
    out = _emb_lookup(idx, embedding_table, pos)
    return out.reshape(_BATCH, _SEQ, _D)


# speedup vs baseline: 2.8773x; 1.9324x over previous
import functools

import numpy as np
import jax
import jax.numpy as jnp
from jax import lax
from jax.experimental import pallas as pl
from jax.experimental.pallas import tpu as pltpu
from jax.experimental.pallas import tpu_sc as plsc

_VOCAB = 100000
_D = 128
_SEQ = 200
_BATCH = 1024
_NW = 32
_CHUNK = 100
_SPW = _BATCH // _NW
_NCH = _SPW * 2
_NBUF = 3
_SCALE = float(np.sqrt(float(_D)))


def _pos_table() -> np.ndarray:
    d = np.arange(_D)
    even = (d % 2 == 0).astype(np.float64)
    odd = (d % 2 == 1).astype(np.float64)
    rate = 1.0 / (10000.0 ** (d[np.newaxis, :] / _D))
    rads = np.arange(_SEQ)[:, np.newaxis] * rate
    return (np.sin(rads) * even + np.cos(rads) * odd).astype(np.float32)


_POS = _pos_table()

_mesh = plsc.VectorSubcoreMesh(core_axis_name="c", subcore_axis_name="s")


@functools.partial(
    pl.kernel,
    mesh=_mesh,
    out_type=jax.ShapeDtypeStruct((_BATCH, _SEQ, _D), jnp.float32),
    scratch_types=[
        pltpu.VMEM((_NCH, _CHUNK), jnp.int32),
        pltpu.VMEM((_SEQ, _D), jnp.float32),
        pltpu.VMEM((_NBUF, _SEQ, _D), jnp.float32),
        pltpu.SemaphoreType.DMA,
        pltpu.SemaphoreType.DMA,
        pltpu.SemaphoreType.DMA,
        pltpu.SemaphoreType.DMA,
        pltpu.SemaphoreType.DMA,
        pltpu.SemaphoreType.DMA,
    ],
)
def _emb_lookup(idx_hbm, tab_hbm, pos_hbm, out_hbm, idx_v, pos_v, buf,
                gs0, gs1, gs2, ss0, ss1, ss2):
    gsems = (gs0, gs1, gs2)
    ssems = (ss0, ss1, ss2)
    wid = lax.axis_index("s") * 2 + lax.axis_index("c")
    b_base = wid * _SPW
    pltpu.sync_copy(pos_hbm, pos_v)
    pltpu.sync_copy(idx_hbm.at[wid], idx_v)

    def issue_gather(q, s):
        pltpu.async_copy(tab_hbm.at[idx_v.at[2 * q]],
                         buf.at[s, pl.ds(0, _CHUNK)], gsems[s])
        pltpu.async_copy(tab_hbm.at[idx_v.at[2 * q + 1]],
                         buf.at[s, pl.ds(_CHUNK, _CHUNK)], gsems[s])

    def wait_gather(s):
        pltpu.make_async_copy(out_hbm.at[0], buf.at[s], gsems[s]).wait()

    def issue_store(q, s):
        pltpu.async_copy(buf.at[s], out_hbm.at[b_base + q], ssems[s])

    def wait_store(s):
        pltpu.make_async_copy(buf.at[s], out_hbm.at[0], ssems[s]).wait()

    def compute(s):
        def row_body(r, c):
            for j in range(_D // 16):
                sl = pl.ds(j * 16, 16)
                buf[s, r, sl] = buf[s, r, sl] * _SCALE + pos_v[r, sl]
            return c

        lax.fori_loop(0, _SEQ, row_body, 0)

    issue_gather(0, 0)
    issue_gather(1, 1)

    wait_gather(0)
    compute(0)
    issue_gather(2, 2)
    issue_store(0, 0)
    for q in (1, 2):
        s = q % _NBUF
        wait_gather(s)
        compute(s)
        wait_store((q - 1) % _NBUF)
        issue_gather(q + 2, (q + 2) % _NBUF)
        issue_store(q, s)

    def block_body(it, carry):
        q0 = it * _NBUF
        for b in range(_NBUF):
            q = q0 + b
            wait_gather(b)
            compute(b)
            wait_store((b - 1) % _NBUF)
            issue_gather(q + 2, (b + 2) % _NBUF)
            issue_store(q, b)
        return carry

    lax.fori_loop(1, _SPW // _NBUF, block_body, 0)

    for q in (_SPW - 2, _SPW - 1):
        s = q % _NBUF
        wait_gather(s)
        compute(s)
        wait_store((q - 1) % _NBUF)
        issue_store(q, s)
    wait_store((_SPW - 1) % _NBUF)


def kernel(x, embedding_table):
    idx = x.reshape(_NW, _NCH, _CHUNK).astype(jnp.int32)
    pos = jnp.asarray(_POS)
    return _emb_lookup(idx, embedding_table, pos)
